# baseline (speedup 1.0000x reference)
import jax
import jax.numpy as jnp
from jax.experimental import pallas as pl
from jax.experimental.pallas import tpu as pltpu

N = 4096
T = 16
BM = 128
NSTEP = N // BM
NBUF = 4


def _fused_kernel(xi, xj1, xk, wi, wj1, wj2, wk,
                  gi_h, aj_h, cj_h, gk_h, out,
                  b_gi, b_aj, b_cj, b_gk, yi, y1, y2, yk, sems):
    bf = jnp.bfloat16
    streams = ((gi_h, b_gi), (aj_h, b_aj), (cj_h, b_cj), (gk_h, b_gk))

    def start_copies(step, slot):
        for m, (hbm, buf) in enumerate(streams):
            pltpu.make_async_copy(
                hbm.at[pl.ds(step * BM, BM), :],
                buf.at[slot],
                sems.at[slot, m],
            ).start()

    for s in range(NBUF):
        start_copies(s, s)

    yi[...] = jnp.dot(
        xi[...], wi[...], preferred_element_type=jnp.float32).astype(bf)
    y1[...] = jnp.dot(
        xj1[...], wj1[...], preferred_element_type=jnp.float32).astype(bf)
    y2[...] = jnp.dot(
        xj1[...], wj2[...], preferred_element_type=jnp.float32).astype(bf)
    yk[...] = jnp.dot(
        xk[...], wk[...], preferred_element_type=jnp.float32).astype(bf)
    ys = (yi, y1, y2, yk)

    for step in range(NSTEP):
        slot = step % NBUF
        for m, (hbm, buf) in enumerate(streams):
            pltpu.make_async_copy(
                hbm.at[pl.ds(step * BM, BM), :],
                buf.at[slot],
                sems.at[slot, m],
            ).wait()
        acc = jnp.zeros((BM, T), dtype=jnp.float32)
        for m, (hbm, buf) in enumerate(streams):
            acc += jnp.dot(buf[slot].astype(bf), ys[m][...],
                           preferred_element_type=jnp.float32)
        out[step * BM:(step + 1) * BM, :] = jnp.maximum(acc, 0.0)
        nxt = step + NBUF
        if nxt < NSTEP:
            start_copies(nxt, slot)


@jax.jit
def kernel(xi, xj1, xj2, xk, Gi2j, Adj2j, coAdj2j, Gk2j, W_i, W_j1, W_j2, W_k):
    del xj2

    vmem_full = pl.BlockSpec(memory_space=pltpu.MemorySpace.VMEM)
    hbm = pl.BlockSpec(memory_space=pl.ANY)
    g_buf = pltpu.VMEM((NBUF, BM, N), jnp.float32)
    y_scratch = pltpu.VMEM((N, T), jnp.bfloat16)
    out = pl.pallas_call(
        _fused_kernel,
        in_specs=[vmem_full, vmem_full, vmem_full,
                  vmem_full, vmem_full, vmem_full, vmem_full,
                  hbm, hbm, hbm, hbm],
        out_specs=vmem_full,
        out_shape=jax.ShapeDtypeStruct((N, T), jnp.float32),
        scratch_shapes=[g_buf, g_buf, g_buf, g_buf,
                        y_scratch, y_scratch, y_scratch, y_scratch,
                        pltpu.SemaphoreType.DMA((NBUF, 4))],
    )(xi, xj1, xk, W_i, W_j1, W_j2, W_k, Gi2j, Adj2j, coAdj2j, Gk2j)
    return out
